# baseline (speedup 1.0000x reference)
import functools

import jax
import jax.numpy as jnp
from jax import lax
from jax.experimental import pallas as pl
from jax.experimental.pallas import tpu as pltpu
from jax.experimental.pallas import tpu_sc as plsc

D = 128
O = 128
KN = 16
NC = 2
NS = 16
NW = NC * NS
RPAD = 20480
NPW = RPAD // NW
G = 8
NB = NPW // G
BM = 1024


def _pre_body(x_ref, pos_ref, w1a_ref, w1b_ref, w1p_ref, b1_ref, c_ref, s_ref):
    xb = x_ref[...]
    posc = jnp.dot(pos_ref[...], w1p_ref[...], preferred_element_type=jnp.float32)
    c_ref[...] = jnp.dot(xb, w1b_ref[...], preferred_element_type=jnp.float32) + posc
    s_ref[...] = (jnp.dot(xb, w1a_ref[...], preferred_element_type=jnp.float32)
                  - posc + b1_ref[...])


def _pre_call(xf, posf, w1a, w1b, w1p, b1r):
    return pl.pallas_call(
        _pre_body,
        grid=(RPAD // BM,),
        in_specs=[
            pl.BlockSpec((BM, D), lambda i: (i, 0)),
            pl.BlockSpec((BM, 8), lambda i: (i, 0)),
            pl.BlockSpec((D, O), lambda i: (0, 0)),
            pl.BlockSpec((D, O), lambda i: (0, 0)),
            pl.BlockSpec((8, O), lambda i: (0, 0)),
            pl.BlockSpec((1, O), lambda i: (0, 0)),
        ],
        out_specs=[
            pl.BlockSpec((BM, O), lambda i: (i, 0)),
            pl.BlockSpec((BM, O), lambda i: (i, 0)),
        ],
        out_shape=[
            jax.ShapeDtypeStruct((RPAD, O), jnp.float32),
            jax.ShapeDtypeStruct((RPAD, O), jnp.float32),
        ],
    )(xf, posf, w1a, w1b, w1p, b1r)


_MESH = plsc.VectorSubcoreMesh(core_axis_name="c", subcore_axis_name="s",
                               num_cores=NC, num_subcores=NS)

SBB = 16
NSB = NB // SBB


@functools.partial(
    pl.kernel,
    mesh=_MESH,
    out_type=jax.ShapeDtypeStruct((RPAD, O), jnp.float32),
    scratch_types=[
        pltpu.VMEM((NB, G * KN), jnp.int32),
        pltpu.VMEM((G * KN, O), jnp.float32),
        pltpu.VMEM((G * KN, O), jnp.float32),
        pltpu.VMEM((G * KN, O), jnp.float32),
        pltpu.VMEM((G * KN, O), jnp.float32),
        pltpu.VMEM((SBB * G, O), jnp.float32),
        pltpu.VMEM((SBB * G, O), jnp.float32),
        pltpu.SemaphoreType.DMA,
        pltpu.SemaphoreType.DMA,
        pltpu.SemaphoreType.DMA,
        pltpu.SemaphoreType.DMA,
    ],
)
def _sc_gather(c_hbm, s_hbm, gidx_hbm, out_hbm, idx_v, r0_v, r1_v, r2_v, r3_v,
               s_v, m_v, sem0, sem1, sem2, sem3):
    wid = lax.axis_index("s") * NC + lax.axis_index("c")
    node0 = wid * NPW
    blk0 = wid * NB
    bufs = (r0_v, r1_v, r2_v, r3_v)
    sems = (sem0, sem1, sem2, sem3)

    pltpu.sync_copy(gidx_hbm.at[pl.ds(blk0, NB)], idx_v)
    for j in range(3):
        pltpu.async_copy(c_hbm.at[idx_v.at[j]], bufs[j], sems[j])

    def compute_block(rows_v, sbase):
        def gbody(g, carry):
            for ch in range(O // 16):
                sv = s_v[sbase + g, pl.ds(ch * 16, 16)]
                nsv = -sv
                acc = jnp.maximum(rows_v[g * KN, pl.ds(ch * 16, 16)], nsv)
                for k in range(1, KN):
                    acc = acc + jnp.maximum(rows_v[g * KN + k, pl.ds(ch * 16, 16)],
                                            nsv)
                m_v[sbase + g, pl.ds(ch * 16, 16)] = acc * (1.0 / KN) + sv
            return carry

        lax.fori_loop(0, G, gbody, 0)

    def sb_body(sb, carry):
        pltpu.sync_copy(s_hbm.at[pl.ds(node0 + sb * (SBB * G), SBB * G)], s_v)

        def quad_body(q, carry2):
            for j in range(4):
                b = sb * SBB + q * 4 + j
                bn = jnp.minimum(b + 3, NB - 1)
                pltpu.async_copy(c_hbm.at[idx_v.at[bn]], bufs[(j + 3) % 4],
                                 sems[(j + 3) % 4])
                pltpu.make_async_copy(c_hbm.at[idx_v.at[b]], bufs[j],
                                      sems[j]).wait()
                compute_block(bufs[j], (q * 4 + j) * G)
            return carry2

        lax.fori_loop(0, SBB // 4, quad_body, 0)
        pltpu.sync_copy(m_v, out_hbm.at[pl.ds(node0 + sb * (SBB * G), SBB * G)])
        return carry

    lax.fori_loop(0, NSB, sb_body, 0)
    for j in range(3):
        pltpu.make_async_copy(c_hbm.at[idx_v.at[0]], bufs[j], sems[j]).wait()


def _post_body(x_ref, m_ref, w2a_ref, w2b_ref, b2_ref, g_ref, be_ref, mk_ref, o_ref):
    h = (jnp.dot(x_ref[...], w2a_ref[...], preferred_element_type=jnp.float32)
         + jnp.dot(m_ref[...], w2b_ref[...], preferred_element_type=jnp.float32)
         + b2_ref[...])
    mu = jnp.mean(h, axis=-1, keepdims=True)
    hc = h - mu
    var = jnp.mean(hc * hc, axis=-1, keepdims=True)
    hn = hc * lax.rsqrt(var + 1e-5) * g_ref[...] + be_ref[...]
    o_ref[...] = jnp.maximum(hn, 0.0) * mk_ref[:, 0:1]


def _post_call(xf, m, w2a, w2b, b2r, gr, ber, mk):
    return pl.pallas_call(
        _post_body,
        grid=(RPAD // BM,),
        in_specs=[
            pl.BlockSpec((BM, D), lambda i: (i, 0)),
            pl.BlockSpec((BM, O), lambda i: (i, 0)),
            pl.BlockSpec((D, O), lambda i: (0, 0)),
            pl.BlockSpec((O, O), lambda i: (0, 0)),
            pl.BlockSpec((1, O), lambda i: (0, 0)),
            pl.BlockSpec((1, O), lambda i: (0, 0)),
            pl.BlockSpec((1, O), lambda i: (0, 0)),
            pl.BlockSpec((BM, 8), lambda i: (i, 0)),
        ],
        out_specs=pl.BlockSpec((BM, O), lambda i: (i, 0)),
        out_shape=jax.ShapeDtypeStruct((RPAD, O), jnp.float32),
    )(xf, m, w2a, w2b, b2r, gr, ber, mk)


def kernel(x, pos, mask, W1, b1, W2, b2, gamma, beta, edge_idx):
    B_, N_, D_ = x.shape
    R = B_ * N_

    xf = jnp.pad(x.reshape(R, D_), ((0, RPAD - R), (0, 0)))
    posf = jnp.pad(pos.reshape(R, 2), ((0, RPAD - R), (0, 6)))
    w1a = W1[:D]
    w1b = W1[D:2 * D]
    w1p = jnp.pad(W1[2 * D:], ((0, 6), (0, 0)))
    b1r = b1.reshape(1, O)

    c, s = _pre_call(xf, posf, w1a, w1b, w1p, b1r)

    gidx = (edge_idx + (jnp.arange(B_, dtype=jnp.int32) * N_)[:, None, None])
    gidx = jnp.pad(gidx.reshape(R * KN), (0, RPAD * KN - R * KN))
    gidx = gidx.reshape(RPAD * KN // (G * KN), G * KN)

    m = _sc_gather(c, s, gidx)

    w2a = W2[:D]
    w2b = W2[D:]
    mk = jnp.pad(mask.reshape(R, 1), ((0, RPAD - R), (0, 7)))
    out = _post_call(xf, m, w2a, w2b, b2.reshape(1, O), gamma.reshape(1, O),
                     beta.reshape(1, O), mk)
    return out[:R].reshape(B_, N_, O)


# speedup vs baseline: 2.1003x; 2.1003x over previous
import functools

import jax
import jax.numpy as jnp
from jax import lax
from jax.experimental import pallas as pl
from jax.experimental.pallas import tpu as pltpu
from jax.experimental.pallas import tpu_sc as plsc

D = 128
O = 128
KN = 16
NC = 2
NS = 16
NW = NC * NS
NPAD = 10240
RPAD = 2 * NPAD
NPW = NPAD // NS
G = 8
NB = NPW // G
BM = 1024
SBB = 16
NSB = NB // SBB


def _pre_body(x_ref, pos_ref, w1a_ref, w1b_ref, w1p_ref, b1_ref, c_ref, s_ref):
    xb = x_ref[...]
    posc = jnp.dot(pos_ref[...], w1p_ref[...], preferred_element_type=jnp.float32)
    c_ref[...] = jnp.dot(xb, w1b_ref[...], preferred_element_type=jnp.float32) + posc
    s_ref[...] = (jnp.dot(xb, w1a_ref[...], preferred_element_type=jnp.float32)
                  - posc + b1_ref[...])


def _pre_call(xf, posf, w1a, w1b, w1p, b1r):
    return pl.pallas_call(
        _pre_body,
        grid=(RPAD // BM,),
        in_specs=[
            pl.BlockSpec((BM, D), lambda i: (i, 0)),
            pl.BlockSpec((BM, 8), lambda i: (i, 0)),
            pl.BlockSpec((D, O), lambda i: (0, 0)),
            pl.BlockSpec((D, O), lambda i: (0, 0)),
            pl.BlockSpec((8, O), lambda i: (0, 0)),
            pl.BlockSpec((1, O), lambda i: (0, 0)),
        ],
        out_specs=[
            pl.BlockSpec((BM, O), lambda i: (i, 0)),
            pl.BlockSpec((BM, O), lambda i: (i, 0)),
        ],
        out_shape=[
            jax.ShapeDtypeStruct((RPAD, O), jnp.float32),
            jax.ShapeDtypeStruct((RPAD, O), jnp.float32),
        ],
    )(xf, posf, w1a, w1b, w1p, b1r)


_MESH = plsc.VectorSubcoreMesh(core_axis_name="c", subcore_axis_name="s",
                               num_cores=NC, num_subcores=NS)


@functools.partial(
    pl.kernel,
    mesh=_MESH,
    out_type=jax.ShapeDtypeStruct((RPAD, O), jnp.float32),
    scratch_types=[
        pltpu.VMEM((NB, G * KN), jnp.int32),
        pltpu.VMEM((G * KN, O), jnp.float32),
        pltpu.VMEM((G * KN, O), jnp.float32),
        pltpu.VMEM((G, O), jnp.float32),
        pltpu.VMEM((G, O), jnp.float32),
        pltpu.VMEM_SHARED((NPAD, O), jnp.float32),
        pltpu.SemaphoreType.DMA,
        pltpu.SemaphoreType.DMA,
    ],
)
def _sc_gather(c_hbm, s_hbm, gidx_hbm, out_hbm, idx_v, r0_v, r1_v,
               s_v, m_v, c_sh, sem0, sem1):
    cid = lax.axis_index("c")
    sid = lax.axis_index("s")
    wid = cid * NS + sid
    node0 = wid * NPW
    blk0 = wid * NB
    pltpu.sync_copy(c_hbm.at[pl.ds(node0, NPW)], c_sh.at[pl.ds(sid * NPW, NPW)])
    pltpu.sync_copy(gidx_hbm.at[pl.ds(blk0, NB)], idx_v)
    plsc.subcore_barrier()

    pltpu.async_copy(c_sh.at[idx_v.at[0]], r0_v, sem0)

    def compute_block(rows_v):
        def gbody(g, carry):
            for ch in range(O // 16):
                sv = s_v[g, pl.ds(ch * 16, 16)]
                nsv = -sv
                acc = jnp.maximum(rows_v[g * KN, pl.ds(ch * 16, 16)], nsv)
                for k in range(1, KN):
                    acc = acc + jnp.maximum(rows_v[g * KN + k, pl.ds(ch * 16, 16)],
                                            nsv)
                m_v[g, pl.ds(ch * 16, 16)] = acc * (1.0 / KN) + sv
            return carry

        lax.fori_loop(0, G, gbody, 0)

    def pair_body(p, carry2):
        b = 2 * p
        nb0 = node0 + b * G
        pltpu.async_copy(c_sh.at[idx_v.at[b + 1]], r1_v, sem1)
        pltpu.sync_copy(s_hbm.at[pl.ds(nb0, G)], s_v)
        pltpu.make_async_copy(c_sh.at[idx_v.at[b]], r0_v, sem0).wait()
        compute_block(r0_v)
        pltpu.sync_copy(m_v, out_hbm.at[pl.ds(nb0, G)])
        bn = jnp.minimum(b + 2, NB - 1)
        pltpu.async_copy(c_sh.at[idx_v.at[bn]], r0_v, sem0)
        pltpu.sync_copy(s_hbm.at[pl.ds(nb0 + G, G)], s_v)
        pltpu.make_async_copy(c_sh.at[idx_v.at[b + 1]], r1_v, sem1).wait()
        compute_block(r1_v)
        pltpu.sync_copy(m_v, out_hbm.at[pl.ds(nb0 + G, G)])
        return carry2

    lax.fori_loop(0, NB // 2, pair_body, 0)
    pltpu.make_async_copy(c_sh.at[idx_v.at[0]], r0_v, sem0).wait()


def _post_body(x_ref, m_ref, w2a_ref, w2b_ref, b2_ref, g_ref, be_ref, mk_ref, o_ref):
    h = (jnp.dot(x_ref[...], w2a_ref[...], preferred_element_type=jnp.float32)
         + jnp.dot(m_ref[...], w2b_ref[...], preferred_element_type=jnp.float32)
         + b2_ref[...])
    mu = jnp.mean(h, axis=-1, keepdims=True)
    hc = h - mu
    var = jnp.mean(hc * hc, axis=-1, keepdims=True)
    hn = hc * lax.rsqrt(var + 1e-5) * g_ref[...] + be_ref[...]
    o_ref[...] = jnp.maximum(hn, 0.0) * mk_ref[:, 0:1]


def _post_call(xf, m, w2a, w2b, b2r, gr, ber, mk):
    return pl.pallas_call(
        _post_body,
        grid=(RPAD // BM,),
        in_specs=[
            pl.BlockSpec((BM, D), lambda i: (i, 0)),
            pl.BlockSpec((BM, O), lambda i: (i, 0)),
            pl.BlockSpec((D, O), lambda i: (0, 0)),
            pl.BlockSpec((O, O), lambda i: (0, 0)),
            pl.BlockSpec((1, O), lambda i: (0, 0)),
            pl.BlockSpec((1, O), lambda i: (0, 0)),
            pl.BlockSpec((1, O), lambda i: (0, 0)),
            pl.BlockSpec((BM, 8), lambda i: (i, 0)),
        ],
        out_specs=pl.BlockSpec((BM, O), lambda i: (i, 0)),
        out_shape=jax.ShapeDtypeStruct((RPAD, O), jnp.float32),
    )(xf, m, w2a, w2b, b2r, gr, ber, mk)


def kernel(x, pos, mask, W1, b1, W2, b2, gamma, beta, edge_idx):
    B_, N_, D_ = x.shape

    xf = jnp.pad(x, ((0, 0), (0, NPAD - N_), (0, 0))).reshape(RPAD, D_)
    posf = jnp.pad(pos, ((0, 0), (0, NPAD - N_), (0, 0)))
    posf = jnp.pad(posf.reshape(RPAD, 2), ((0, 0), (0, 6)))
    w1a = W1[:D]
    w1b = W1[D:2 * D]
    w1p = jnp.pad(W1[2 * D:], ((0, 6), (0, 0)))
    b1r = b1.reshape(1, O)

    c, s = _pre_call(xf, posf, w1a, w1b, w1p, b1r)

    gidx = jnp.pad(edge_idx, ((0, 0), (0, NPAD - N_), (0, 0)))
    gidx = gidx.reshape(RPAD * KN // (G * KN), G * KN)

    m = _sc_gather(c, s, gidx)

    w2a = W2[:D]
    w2b = W2[D:]
    mk = jnp.pad(mask, ((0, 0), (0, NPAD - N_)))
    mk = jnp.pad(mk.reshape(RPAD, 1), ((0, 0), (0, 7)))
    out = _post_call(xf, m, w2a, w2b, b2.reshape(1, O), gamma.reshape(1, O),
                     beta.reshape(1, O), mk)
    return out.reshape(B_, NPAD, O)[:, :N_]
